# baseline (speedup 1.0000x reference)
import numpy as np
import jax
import jax.numpy as jnp
from jax.experimental import pallas as pl
from jax.experimental.pallas import tpu as pltpu

_T = 256
_N = 16384
_R = 128
_TS = 32
_MAX_ITERS = 40


def _tf_block_np(k0, k1, x0, x1):
    k0 = np.uint32(k0)
    k1 = np.uint32(k1)
    x0 = np.asarray(x0, np.uint32).copy()
    x1 = np.asarray(x1, np.uint32).copy()
    ks = [k0, k1, np.uint32(k0 ^ k1 ^ np.uint32(0x1BD11BDA))]
    rotations = [[13, 15, 26, 6], [17, 29, 16, 24]]
    x0 = (x0 + ks[0]).astype(np.uint32)
    x1 = (x1 + ks[1]).astype(np.uint32)
    for g in range(5):
        for r in rotations[g % 2]:
            x0 = (x0 + x1).astype(np.uint32)
            x1 = ((x1 << np.uint32(r)) | (x1 >> np.uint32(32 - r))).astype(np.uint32)
            x1 = x1 ^ x0
        x0 = (x0 + ks[(g + 1) % 3]).astype(np.uint32)
        x1 = (x1 + ks[(g + 2) % 3] + np.uint32(g + 1)).astype(np.uint32)
    return x0, x1


def _subkey_schedule(n_iters):
    a, b = _tf_block_np(0, 0, np.uint32([0]), np.uint32([1]))
    rng = (a[0], b[0])
    out = np.empty((n_iters, 2), np.uint32)
    for i in range(n_iters):
        b1, b2 = _tf_block_np(rng[0], rng[1], np.uint32([0, 0]), np.uint32([0, 1]))
        out[i, 0], out[i, 1] = b1[1], b2[1]
        rng = (b1[0], b2[0])
    return out.view(np.int32)


_SUBKEYS = _subkey_schedule(_MAX_ITERS)
_KS_PARITY = np.int32(np.uint32(0x1BD11BDA))


def _tf_bits(k0, k1, x1):
    srl = jax.lax.shift_right_logical

    def rotl(x, r):
        return jax.lax.shift_left(x, r) | srl(x, 32 - r)

    ks0, ks1 = k0, k1
    ks2 = k0 ^ k1 ^ _KS_PARITY
    ks = (ks0, ks1, ks2)
    rotations = ((13, 15, 26, 6), (17, 29, 16, 24))
    x0 = jnp.zeros_like(x1) + ks0
    x1 = x1 + ks1
    for g in range(5):
        for r in rotations[g % 2]:
            x0 = x0 + x1
            x1 = rotl(x1, r)
            x1 = x1 ^ x0
        x0 = x0 + ks[(g + 1) % 3]
        x1 = x1 + ks[(g + 2) % 3] + np.int32(g + 1)
    return x0 ^ x1


def _spike_kernel(img_ref, keys_ref, out_ref):
    lam = img_ref[...]
    sub = jax.lax.broadcasted_iota(jnp.int32, (_TS, _R), 0)
    lane = jax.lax.broadcasted_iota(jnp.int32, (_TS, _R), 1)
    i0 = pl.program_id(0) * _R
    base = (i0 + lane) * _T + sub
    zero_i = jax.lax.shift_right_logical(base, 31)
    zero_f = zero_i.astype(jnp.float32)
    neg_lam = zero_f - jnp.broadcast_to(lam[None, :], (_TS, _R))
    fill = jnp.int32(-2147483648)
    run = jnp.full((1, _R), fill, jnp.int32)

    for s in range(_T // _TS):
        cnt = base + np.int32(s * _TS)

        def cond(carry):
            m, _, logp = carry
            return jnp.logical_and(m < _MAX_ITERS, jnp.any(logp > neg_lam))

        def body(carry):
            m, k, logp = carry
            k0 = keys_ref[m, 0]
            k1 = keys_ref[m, 1]
            k = k + (logp > neg_lam).astype(jnp.int32)
            bits = _tf_bits(k0, k1, cnt)
            fb = jax.lax.shift_right_logical(bits, 9) | np.int32(0x3F800000)
            u = jax.lax.bitcast_convert_type(fb, jnp.float32) - 1.0
            logp = logp + jnp.log(u)
            return m + 1, k, logp

        init = (jnp.int32(0), zero_i, zero_f)
        _, k, _ = jax.lax.while_loop(cond, body, init)

        jsub = sub + np.int32(s * _TS)
        x = jsub + k - 1
        d = 1
        while d < _TS:
            shifted = jnp.concatenate(
                [jnp.full((d, _R), fill, jnp.int32), x[: _TS - d]], axis=0)
            x = jnp.maximum(x, shifted)
            d *= 2
        x = jnp.maximum(x, jnp.broadcast_to(run, (_TS, _R)))
        run = x[_TS - 1:_TS]
        out_ref[pl.dslice(s * _TS, _TS), :] = x > jsub


def kernel(img):
    keys = jnp.asarray(_SUBKEYS)
    out = pl.pallas_call(
        _spike_kernel,
        grid=(_N // _R,),
        in_specs=[
            pl.BlockSpec((_R,), lambda g: (g,)),
            pl.BlockSpec(memory_space=pltpu.SMEM),
        ],
        out_specs=pl.BlockSpec((_T, _R), lambda g: (0, g)),

        out_shape=jax.ShapeDtypeStruct((_T, _N), jnp.bool_),
        compiler_params=pltpu.CompilerParams(
            dimension_semantics=("parallel",),
        ),
    )(img, keys)
    return out


# speedup vs baseline: 2.1163x; 2.1163x over previous
import numpy as np
import jax
import jax.numpy as jnp
from jax.experimental import pallas as pl
from jax.experimental.pallas import tpu as pltpu

_T = 256
_N = 16384
_R = 256
_TS = 64
_MAX_ITERS = 40


def _tf_block_np(k0, k1, x0, x1):
    k0 = np.uint32(k0)
    k1 = np.uint32(k1)
    x0 = np.asarray(x0, np.uint32).copy()
    x1 = np.asarray(x1, np.uint32).copy()
    ks = [k0, k1, np.uint32(k0 ^ k1 ^ np.uint32(0x1BD11BDA))]
    rotations = [[13, 15, 26, 6], [17, 29, 16, 24]]
    x0 = (x0 + ks[0]).astype(np.uint32)
    x1 = (x1 + ks[1]).astype(np.uint32)
    for g in range(5):
        for r in rotations[g % 2]:
            x0 = (x0 + x1).astype(np.uint32)
            x1 = ((x1 << np.uint32(r)) | (x1 >> np.uint32(32 - r))).astype(np.uint32)
            x1 = x1 ^ x0
        x0 = (x0 + ks[(g + 1) % 3]).astype(np.uint32)
        x1 = (x1 + ks[(g + 2) % 3] + np.uint32(g + 1)).astype(np.uint32)
    return x0, x1


def _subkey_schedule(n_iters):
    a, b = _tf_block_np(0, 0, np.uint32([0]), np.uint32([1]))
    rng = (a[0], b[0])
    out = np.empty((n_iters, 2), np.uint32)
    for i in range(n_iters):
        b1, b2 = _tf_block_np(rng[0], rng[1], np.uint32([0, 0]), np.uint32([0, 1]))
        out[i, 0], out[i, 1] = b1[1], b2[1]
        rng = (b1[0], b2[0])
    return out.view(np.int32)


_SUBKEYS = _subkey_schedule(_MAX_ITERS)
_KS_PARITY = np.int32(np.uint32(0x1BD11BDA))


def _tf_bits(k0, k1, x1):
    srl = jax.lax.shift_right_logical

    def rotl(x, r):
        return jax.lax.shift_left(x, r) | srl(x, 32 - r)

    ks0, ks1 = k0, k1
    ks2 = k0 ^ k1 ^ _KS_PARITY
    ks = (ks0, ks1, ks2)
    rotations = ((13, 15, 26, 6), (17, 29, 16, 24))
    x0 = jnp.zeros_like(x1) + ks0
    x1 = x1 + ks1
    for g in range(5):
        for r in rotations[g % 2]:
            x0 = x0 + x1
            x1 = rotl(x1, r)
            x1 = x1 ^ x0
        x0 = x0 + ks[(g + 1) % 3]
        x1 = x1 + ks[(g + 2) % 3] + np.int32(g + 1)
    return x0 ^ x1


def _spike_kernel(img_ref, keys_ref, out_ref):
    lam = img_ref[...]
    sub = jax.lax.broadcasted_iota(jnp.int32, (_TS, _R), 0)
    lane = jax.lax.broadcasted_iota(jnp.int32, (_TS, _R), 1)
    i0 = pl.program_id(0) * _R
    base = (i0 + lane) * _T + sub
    zero_i = jax.lax.shift_right_logical(base, 31)
    zero_f = zero_i.astype(jnp.float32)
    neg_lam = zero_f - jnp.broadcast_to(lam[None, :], (_TS, _R))
    fill = jnp.int32(-2147483648)
    run = jnp.full((1, _R), fill, jnp.int32)

    for s in range(_T // _TS):
        cnt = base + np.int32(s * _TS)

        def cond(carry):
            m, _, logp = carry
            return jnp.logical_and(m < _MAX_ITERS, jnp.any(logp > neg_lam))

        def body(carry):
            m, k, logp = carry
            k0 = keys_ref[m, 0]
            k1 = keys_ref[m, 1]
            k = k + (logp > neg_lam).astype(jnp.int32)
            bits = _tf_bits(k0, k1, cnt)
            fb = jax.lax.shift_right_logical(bits, 9) | np.int32(0x3F800000)
            u = jax.lax.bitcast_convert_type(fb, jnp.float32) - 1.0
            logp = logp + jnp.log(u)
            return m + 1, k, logp

        init = (jnp.int32(0), zero_i, zero_f)
        _, k, _ = jax.lax.while_loop(cond, body, init)

        jsub = sub + np.int32(s * _TS)
        x = jsub + k - 1
        d = 1
        while d < _TS:
            shifted = jnp.concatenate(
                [jnp.full((d, _R), fill, jnp.int32), x[: _TS - d]], axis=0)
            x = jnp.maximum(x, shifted)
            d *= 2
        x = jnp.maximum(x, jnp.broadcast_to(run, (_TS, _R)))
        run = x[_TS - 1:_TS]
        out_ref[pl.dslice(s * _TS, _TS), :] = x > jsub


def kernel(img):
    keys = jnp.asarray(_SUBKEYS)
    out = pl.pallas_call(
        _spike_kernel,
        grid=(_N // _R,),
        in_specs=[
            pl.BlockSpec((_R,), lambda g: (g,)),
            pl.BlockSpec(memory_space=pltpu.SMEM),
        ],
        out_specs=pl.BlockSpec((_T, _R), lambda g: (0, g)),

        out_shape=jax.ShapeDtypeStruct((_T, _N), jnp.bool_),
        compiler_params=pltpu.CompilerParams(
            dimension_semantics=("parallel",),
        ),
    )(img, keys)
    return out
